# baseline (speedup 1.0000x reference)
import functools

import jax
import jax.numpy as jnp
from jax import lax
from jax.experimental import pallas as pl
from jax.experimental.pallas import tpu as pltpu
from jax.experimental.pallas import tpu_sc as plsc

D = 64
L = 200
B = 4096
NC, NS = 2, 16
NW = NC * NS
R = B // NW
HALF = L // 2
HP = 104


def _sc_body(x_hbm, w_hbm, b_hbm, table_hbm, out_hbm,
             idx_v, w_v, b_v, rows0, rows1, out_v, sem0, sem1):
    cid = lax.axis_index("c")
    sid = lax.axis_index("s")
    wid = sid * NC + cid
    base = wid * R

    pltpu.sync_copy(w_hbm, w_v)
    pltpu.sync_copy(b_hbm, b_v)
    pltpu.sync_copy(x_hbm.at[pl.ds(base, R)], idx_v)

    def fire(r, rows_buf, sem):
        for j in range(2):
            pltpu.make_async_copy(
                table_hbm.at[idx_v.at[r, j]], rows_buf.at[j], sem).start()

    def drain(r, rows_buf, sem):
        for j in range(2):
            pltpu.make_async_copy(
                table_hbm.at[idx_v.at[r, j]], rows_buf.at[j], sem).wait()

    b_vec = b_v[pl.ds(0, 16)]
    lanes = lax.broadcasted_iota(jnp.int32, (16,), 0)

    def allreduce16(v):
        dnums = lax.GatherDimensionNumbers(
            offset_dims=(), collapsed_slice_dims=(0,), start_index_map=(0,))
        for k in (1, 2, 4, 8):
            perm = jnp.bitwise_xor(lanes, k)
            v = v + lax.gather(v, perm[:, None], dnums, slice_sizes=(1,),
                               mode=lax.GatherScatterMode.PROMISE_IN_BOUNDS)
        return v

    def compute(rows_buf):
        def body_i(i, accs):
            a = list(accs)
            for j in range(2):
                l = j * HALF + i
                for k in range(4):
                    rv = rows_buf[j, i, pl.ds(k * 16, 16)]
                    wv = w_v[l, pl.ds(k * 16, 16)]
                    a[k] = a[k] + rv * wv
            return tuple(a)

        init = tuple(jnp.zeros((16,), jnp.float32) for _ in range(4))
        a0, a1, a2, a3 = lax.fori_loop(0, HALF, body_i, init)
        tot = (a0 + a1) + (a2 + a3)
        return allreduce16(tot) + b_vec

    fire(0, rows0, sem0)
    fire(1, rows1, sem1)

    def loop_body(t, vec):
        g = t * 2
        drain(g, rows0, sem0)
        s0 = compute(rows0)
        fire(lax.min(g + 2, R - 1), rows0, sem0)
        vec = jnp.where(lanes == g % 16, s0, vec)
        drain(g + 1, rows1, sem1)
        s1 = compute(rows1)
        fire(lax.min(g + 3, R - 1), rows1, sem1)
        vec = jnp.where(lanes == (g + 1) % 16, s1, vec)

        @pl.when(t % 8 == 7)
        def _():
            out_v[pl.ds(g - 14, 16)] = vec

        return vec

    lax.fori_loop(0, R // 2, loop_body, jnp.zeros((16,), jnp.float32))

    drain(R - 1, rows0, sem0)
    drain(R - 1, rows1, sem1)

    pltpu.sync_copy(out_v, out_hbm.at[pl.ds(base, R)])


@functools.partial(jax.jit, static_argnames=())
def kernel(x, table, W, b):
    x3 = x.reshape(B, 2, HALF).astype(jnp.int32)
    x3 = jnp.pad(x3, ((0, 0), (0, 0), (0, HP - HALF)))
    w2 = W.reshape(L, D).astype(jnp.float32)
    b16 = jnp.broadcast_to(b.astype(jnp.float32), (16,))

    mesh = plsc.VectorSubcoreMesh(core_axis_name="c", subcore_axis_name="s")
    call = functools.partial(
        pl.kernel,
        mesh=mesh,
        out_type=jax.ShapeDtypeStruct((B,), jnp.float32),
        compiler_params=pltpu.CompilerParams(use_tc_tiling_on_sc=False),
        scratch_types=[
            pltpu.VMEM((R, 2, HP), jnp.int32),
            pltpu.VMEM((L, D), jnp.float32),
            pltpu.VMEM((16,), jnp.float32),
            pltpu.VMEM((2, HP, D), jnp.float32),
            pltpu.VMEM((2, HP, D), jnp.float32),
            pltpu.VMEM((R,), jnp.float32),
            pltpu.SemaphoreType.DMA,
            pltpu.SemaphoreType.DMA,
        ],
    )(_sc_body)
    return call(x3, w2, b16, table)
